# baseline (speedup 1.0000x reference)
import functools

import jax
import jax.numpy as jnp
from jax import lax
from jax.experimental import pallas as pl
from jax.experimental.pallas import tpu as pltpu
from jax.experimental.pallas import tpu_sc as plsc

VOCAB = 100000
D = 32
B = 1024

_info = plsc.get_sparse_core_info()
_NC, _NS = _info.num_cores, _info.num_subcores
_NW = _NC * _NS
_B_PER_W = B // _NW

_sc_mesh = plsc.VectorSubcoreMesh(core_axis_name="c", subcore_axis_name="s")


@functools.partial(
    pl.kernel,
    mesh=_sc_mesh,
    out_type=jax.ShapeDtypeStruct((B, D), jnp.float32),
    scratch_types=[
        pltpu.VMEM((_B_PER_W,), jnp.int32),
        pltpu.VMEM((_B_PER_W, D), jnp.float32),
        pltpu.SemaphoreType.DMA,
    ],
    compiler_params=pltpu.CompilerParams(use_tc_tiling_on_sc=False),
)
def _sc_gather(table_hbm, idx_hbm, out_hbm, idx_v, rows_v, sem):
    wid = lax.axis_index("s") * _NC + lax.axis_index("c")
    base = wid * _B_PER_W
    pltpu.sync_copy(idx_hbm.at[pl.ds(base, _B_PER_W)], idx_v)
    pltpu.async_copy(table_hbm.at[idx_v], rows_v, sem).wait()
    pltpu.sync_copy(rows_v, out_hbm.at[pl.ds(base, _B_PER_W)])


def _matmul_body(e_ref, p_ref, o_ref):
    o_ref[...] = lax.dot_general(
        e_ref[...],
        p_ref[...],
        (((1,), (1,)), ((), ())),
        preferred_element_type=jnp.float32,
    )


def _projection(e, proj_weight, block_n):
    n_blocks = pl.cdiv(VOCAB, block_n)
    return pl.pallas_call(
        _matmul_body,
        grid=(n_blocks,),
        in_specs=[
            pl.BlockSpec((B, D), lambda i: (0, 0)),
            pl.BlockSpec((block_n, D), lambda i: (i, 0)),
        ],
        out_specs=pl.BlockSpec((B, block_n), lambda i: (0, i)),
        out_shape=jax.ShapeDtypeStruct((B, VOCAB), jnp.float32),
        compiler_params=pltpu.CompilerParams(
            dimension_semantics=("parallel",),
        ),
    )(e, proj_weight)


def kernel(x, embed_weight, proj_weight):
    e = _sc_gather(embed_weight, x.astype(jnp.int32))
    return _projection(e, proj_weight, block_n=2048)


# speedup vs baseline: 1.0272x; 1.0272x over previous
import functools

import jax
import jax.numpy as jnp
from jax import lax
from jax.experimental import pallas as pl
from jax.experimental.pallas import tpu as pltpu
from jax.experimental.pallas import tpu_sc as plsc

VOCAB = 100000
D = 32
B = 1024

_info = plsc.get_sparse_core_info()
_NC, _NS = _info.num_cores, _info.num_subcores
_NW = _NC * _NS
_B_PER_W = B // _NW

_sc_mesh = plsc.VectorSubcoreMesh(core_axis_name="c", subcore_axis_name="s")


@functools.partial(
    pl.kernel,
    mesh=_sc_mesh,
    out_type=jax.ShapeDtypeStruct((B, D), jnp.float32),
    scratch_types=[
        pltpu.VMEM((_B_PER_W,), jnp.int32),
        pltpu.VMEM((_B_PER_W, D), jnp.float32),
        pltpu.SemaphoreType.DMA,
    ],
    compiler_params=pltpu.CompilerParams(use_tc_tiling_on_sc=False),
)
def _sc_gather(table_hbm, idx_hbm, out_hbm, idx_v, rows_v, sem):
    wid = lax.axis_index("s") * _NC + lax.axis_index("c")
    base = wid * _B_PER_W
    pltpu.sync_copy(idx_hbm.at[pl.ds(base, _B_PER_W)], idx_v)
    pltpu.async_copy(table_hbm.at[idx_v], rows_v, sem).wait()
    pltpu.sync_copy(rows_v, out_hbm.at[pl.ds(base, _B_PER_W)])


def _matmul_body(e_ref, p_ref, o_ref):
    o_ref[...] = lax.dot_general(
        e_ref[...],
        p_ref[...],
        (((1,), (1,)), ((), ())),
        preferred_element_type=jnp.float32,
    )


def _projection(e, proj_weight, block_n):
    n_blocks = pl.cdiv(VOCAB, block_n)
    return pl.pallas_call(
        _matmul_body,
        grid=(n_blocks,),
        in_specs=[
            pl.BlockSpec((B, D), lambda i: (0, 0)),
            pl.BlockSpec((block_n, D), lambda i: (i, 0)),
        ],
        out_specs=pl.BlockSpec((B, block_n), lambda i: (0, i)),
        out_shape=jax.ShapeDtypeStruct((B, VOCAB), jnp.float32),
        compiler_params=pltpu.CompilerParams(
            dimension_semantics=("parallel",),
        ),
    )(e, proj_weight)


def kernel(x, embed_weight, proj_weight):
    e = _sc_gather(embed_weight, x.astype(jnp.int32))
    return _projection(e, proj_weight, block_n=4096)


# speedup vs baseline: 1.0483x; 1.0206x over previous
import functools

import jax
import jax.numpy as jnp
from jax import lax
from jax.experimental import pallas as pl
from jax.experimental.pallas import tpu as pltpu
from jax.experimental.pallas import tpu_sc as plsc

VOCAB = 100000
D = 32
B = 1024

_info = plsc.get_sparse_core_info()
_NC, _NS = _info.num_cores, _info.num_subcores
_NW = _NC * _NS
_B_PER_W = B // _NW

_sc_mesh = plsc.VectorSubcoreMesh(core_axis_name="c", subcore_axis_name="s")


@functools.partial(
    pl.kernel,
    mesh=_sc_mesh,
    out_type=jax.ShapeDtypeStruct((B, D), jnp.float32),
    scratch_types=[
        pltpu.VMEM((_B_PER_W,), jnp.int32),
        pltpu.VMEM((_B_PER_W, D), jnp.float32),
        pltpu.SemaphoreType.DMA,
    ],
    compiler_params=pltpu.CompilerParams(use_tc_tiling_on_sc=False),
)
def _sc_gather(table_hbm, idx_hbm, out_hbm, idx_v, rows_v, sem):
    wid = lax.axis_index("s") * _NC + lax.axis_index("c")
    base = wid * _B_PER_W
    pltpu.sync_copy(idx_hbm.at[pl.ds(base, _B_PER_W)], idx_v)
    pltpu.async_copy(table_hbm.at[idx_v], rows_v, sem).wait()
    pltpu.sync_copy(rows_v, out_hbm.at[pl.ds(base, _B_PER_W)])


def _matmul_body(e_ref, p_ref, o_ref):
    o_ref[...] = lax.dot_general(
        e_ref[...],
        p_ref[...],
        (((1,), (1,)), ((), ())),
        preferred_element_type=jnp.float32,
    )


def _projection(e, proj_weight, block_n):
    n_blocks = pl.cdiv(VOCAB, block_n)
    return pl.pallas_call(
        _matmul_body,
        grid=(n_blocks,),
        in_specs=[
            pl.BlockSpec((B, D), lambda i: (0, 0)),
            pl.BlockSpec((block_n, D), lambda i: (i, 0)),
        ],
        out_specs=pl.BlockSpec((B, block_n), lambda i: (0, i)),
        out_shape=jax.ShapeDtypeStruct((B, VOCAB), jnp.float32),
        compiler_params=pltpu.CompilerParams(
            dimension_semantics=("parallel",),
        ),
    )(e, proj_weight)


def kernel(x, embed_weight, proj_weight):
    e = jnp.take(embed_weight, x, axis=0)
    return _projection(e, proj_weight, block_n=4096)


# speedup vs baseline: 3.5104x; 3.3485x over previous
import functools

import jax
import jax.numpy as jnp
from jax import lax
from jax.experimental import pallas as pl
from jax.experimental.pallas import tpu as pltpu
from jax.experimental.pallas import tpu_sc as plsc

VOCAB = 100000
D = 32
B = 1024

_info = plsc.get_sparse_core_info()
_NC, _NS = _info.num_cores, _info.num_subcores
_NW = _NC * _NS
_IDX_ROWS = B // 128

_sc_mesh = plsc.VectorSubcoreMesh(core_axis_name="c", subcore_axis_name="s")


@functools.partial(
    pl.kernel,
    mesh=_sc_mesh,
    out_type=jax.ShapeDtypeStruct((D, B), jnp.float32),
    scratch_types=[
        pltpu.VMEM((B,), jnp.int32),
        pltpu.VMEM((_IDX_ROWS, 128), jnp.int32),
        pltpu.VMEM((B,), jnp.float32),
        pltpu.SemaphoreType.DMA,
    ],
    compiler_params=pltpu.CompilerParams(use_tc_tiling_on_sc=False),
)
def _sc_gather(table_hbm, idx_hbm, out_hbm, x_v, idx_v, vals_v, sem):
    d = lax.axis_index("s") * _NC + lax.axis_index("c")
    pltpu.sync_copy(idx_hbm, x_v)
    base = d * VOCAB
    for r in range(_IDX_ROWS):
        for c in range(128 // 16):
            xv = x_v[pl.ds(r * 128 + c * 16, 16)]
            idx_v[r, pl.ds(c * 16, 16)] = xv + base
    copies = [
        pltpu.async_copy(
            table_hbm.at[idx_v.at[r]], vals_v.at[pl.ds(r * 128, 128)], sem
        )
        for r in range(_IDX_ROWS)
    ]
    for cp in copies:
        cp.wait()
    pltpu.sync_copy(vals_v, out_hbm.at[d])


def _matmul_body(p_ref, e_ref, o_ref):
    o_ref[...] = lax.dot_general(
        p_ref[...],
        e_ref[...],
        (((0,), (0,)), ((), ())),
        preferred_element_type=jnp.float32,
    )


def _projection_t(proj_t, e_t, block_m):
    n_blocks = pl.cdiv(VOCAB, block_m)
    return pl.pallas_call(
        _matmul_body,
        grid=(n_blocks,),
        in_specs=[
            pl.BlockSpec((D, block_m), lambda i: (0, i)),
            pl.BlockSpec((D, B), lambda i: (0, 0)),
        ],
        out_specs=pl.BlockSpec((block_m, B), lambda i: (i, 0)),
        out_shape=jax.ShapeDtypeStruct((VOCAB, B), jnp.float32),
        compiler_params=pltpu.CompilerParams(
            dimension_semantics=("parallel",),
        ),
    )(proj_t, e_t)


def kernel(x, embed_weight, proj_weight):
    table_flat = embed_weight.T.reshape(-1)
    e_t = _sc_gather(table_flat, x.astype(jnp.int32))
    out_t = _projection_t(proj_weight.T, e_t, block_m=2048)
    return out_t.T


# speedup vs baseline: 3.8510x; 1.0970x over previous
import functools

import jax
import jax.numpy as jnp
from jax import lax
from jax.experimental import pallas as pl
from jax.experimental.pallas import tpu as pltpu
from jax.experimental.pallas import tpu_sc as plsc

VOCAB = 100000
D = 32
B = 1024

_info = plsc.get_sparse_core_info()
_NC, _NS = _info.num_cores, _info.num_subcores
_NW = _NC * _NS
assert _NW == D, "one vector subcore per embedding dimension"

_sc_mesh = plsc.VectorSubcoreMesh(core_axis_name="c", subcore_axis_name="s")


@functools.partial(
    pl.kernel,
    mesh=_sc_mesh,
    out_type=jax.ShapeDtypeStruct((D, B), jnp.float32),
    scratch_types=[
        pltpu.VMEM((B,), jnp.int32),
        pltpu.VMEM((VOCAB,), jnp.float32),
        pltpu.VMEM((B,), jnp.float32),
    ],
    compiler_params=pltpu.CompilerParams(
        use_tc_tiling_on_sc=True, needs_layout_passes=False, skip_device_barrier=True
    ),
)
def _sc_gather(table_hbm, idx_hbm, out_hbm, x_v, row_v, vals_v):
    d = lax.axis_index("s") * _NC + lax.axis_index("c")
    pltpu.sync_copy(idx_hbm, x_v)
    pltpu.sync_copy(table_hbm.at[d], row_v)
    for c in range(B // 16):
        idx16 = x_v[pl.ds(c * 16, 16)]
        vals_v[pl.ds(c * 16, 16)] = plsc.load_gather(row_v, [idx16])
    pltpu.sync_copy(vals_v, out_hbm.at[d])


def _matmul_body(p_ref, e_ref, o_ref):
    o_ref[...] = lax.dot_general(
        p_ref[...],
        e_ref[...],
        (((0,), (0,)), ((), ())),
        preferred_element_type=jnp.float32,
    )


def _projection_t(proj_t, e_t, block_m):
    n_blocks = pl.cdiv(VOCAB, block_m)
    return pl.pallas_call(
        _matmul_body,
        grid=(n_blocks,),
        in_specs=[
            pl.BlockSpec((D, block_m), lambda i: (0, i)),
            pl.BlockSpec((D, B), lambda i: (0, 0)),
        ],
        out_specs=pl.BlockSpec((block_m, B), lambda i: (i, 0)),
        out_shape=jax.ShapeDtypeStruct((VOCAB, B), jnp.float32),
        compiler_params=pltpu.CompilerParams(
            dimension_semantics=("parallel",),
            vmem_limit_bytes=128 * 1024 * 1024,
        ),
    )(proj_t, e_t)


def kernel(x, embed_weight, proj_weight):
    e_t = _sc_gather(embed_weight.T, x.astype(jnp.int32))
    out_t = _projection_t(proj_weight.T, e_t, block_m=4096)
    return out_t.T


# speedup vs baseline: 3.8542x; 1.0008x over previous
import functools

import jax
import jax.numpy as jnp
from jax import lax
from jax.experimental import pallas as pl
from jax.experimental.pallas import tpu as pltpu
from jax.experimental.pallas import tpu_sc as plsc

VOCAB = 100000
D = 32
B = 1024

_info = plsc.get_sparse_core_info()
_NC, _NS = _info.num_cores, _info.num_subcores
_NW = _NC * _NS
assert _NW == D, "one vector subcore per embedding dimension"

_sc_mesh = plsc.VectorSubcoreMesh(core_axis_name="c", subcore_axis_name="s")


@functools.partial(
    pl.kernel,
    mesh=_sc_mesh,
    out_type=jax.ShapeDtypeStruct((D, B), jnp.float32),
    scratch_types=[
        pltpu.VMEM((B,), jnp.int32),
        pltpu.VMEM((VOCAB,), jnp.float32),
        pltpu.VMEM((B,), jnp.float32),
    ],
    compiler_params=pltpu.CompilerParams(
        use_tc_tiling_on_sc=True, needs_layout_passes=False, skip_device_barrier=True
    ),
)
def _sc_gather(table_hbm, idx_hbm, out_hbm, x_v, row_v, vals_v):
    d = lax.axis_index("s") * _NC + lax.axis_index("c")
    pltpu.sync_copy(idx_hbm, x_v)
    pltpu.sync_copy(table_hbm.at[d], row_v)
    for c in range(B // 16):
        idx16 = x_v[pl.ds(c * 16, 16)]
        vals_v[pl.ds(c * 16, 16)] = plsc.load_gather(row_v, [idx16])
    pltpu.sync_copy(vals_v, out_hbm.at[d])


def _matmul_body(p_ref, e_ref, o_ref):
    o_ref[...] = lax.dot_general(
        p_ref[...],
        e_ref[...],
        (((0,), (0,)), ((), ())),
        preferred_element_type=jnp.float32,
    )


def _projection_t(proj_t, e_t, block_m):
    n_blocks = pl.cdiv(VOCAB, block_m)
    return pl.pallas_call(
        _matmul_body,
        grid=(n_blocks,),
        in_specs=[
            pl.BlockSpec((D, block_m), lambda i: (0, i)),
            pl.BlockSpec((D, B), lambda i: (0, 0)),
        ],
        out_specs=pl.BlockSpec((block_m, B), lambda i: (i, 0)),
        out_shape=jax.ShapeDtypeStruct((VOCAB, B), jnp.float32),
        compiler_params=pltpu.CompilerParams(
            dimension_semantics=("parallel",),
            vmem_limit_bytes=128 * 1024 * 1024,
            skip_device_barrier=True,
        ),
    )(proj_t, e_t)


def kernel(x, embed_weight, proj_weight):
    e_t = _sc_gather(embed_weight.T, x.astype(jnp.int32))
    out_t = _projection_t(proj_weight.T, e_t, block_m=4096)
    return out_t.T
